# baseline (speedup 1.0000x reference)
import functools

import jax
import jax.numpy as jnp
from jax import lax
from jax.experimental import pallas as pl
from jax.experimental.pallas import tpu as pltpu
from jax.experimental.pallas import tpu_sc as plsc

N_NODES = 10000
N_EDGES = 160000
D = 256

NP = 10240
NC = 2
NS = 16
NW = NC * NS
RT = NP // NW
K = 112
ES = N_EDGES + 160
BM = 1024


def _sc_agg(lin, srcs, dsts, ws, bounds):
    mesh = plsc.VectorSubcoreMesh(core_axis_name="c", subcore_axis_name="s")

    @functools.partial(
        pl.kernel,
        out_type=jax.ShapeDtypeStruct((NP, D), jnp.float32),
        mesh=mesh,
        scratch_types=[
            pltpu.VMEM((48,), jnp.int32),
            pltpu.VMEM((K,), jnp.int32),
            pltpu.VMEM((K + 16,), jnp.int32),
            pltpu.VMEM((K,), jnp.float32),
            pltpu.VMEM((K, D), jnp.float32),
            pltpu.VMEM((RT, D), jnp.float32),
            pltpu.SemaphoreType.DMA,
        ],
    )
    def k(lin_hbm, srcs_hbm, dsts_hbm, ws_hbm, bounds_hbm, out_hbm,
          bounds_v, srcc, dstc, wc, rows_v, acc, sem):
        c = lax.axis_index("c")
        s = lax.axis_index("s")
        t = c * NS + s
        lo = t * RT
        lanes = lax.iota(jnp.int32, 16)
        zv = jnp.zeros((16,), jnp.float32)

        def zero_row(r, _):
            for ci in range(D // 16):
                acc[r, pl.ds(ci * 16, 16)] = zv
            return 0

        lax.fori_loop(0, RT, zero_row, 0)

        pltpu.sync_copy(bounds_hbm, bounds_v)

        b0 = bounds_v[pl.ds(t, 16)][0]
        b1 = bounds_v[pl.ds(t + 1, 16)][0]
        start = (b0 // 8) * 8
        nch = (b1 - start + K - 1) // K

        def chunk_body(jc, _):
            off = start + jc * K
            pltpu.sync_copy(srcs_hbm.at[pl.ds(off, K)], srcc)
            pltpu.sync_copy(dsts_hbm.at[pl.ds(off, K)], dstc.at[pl.ds(0, K)])
            pltpu.sync_copy(ws_hbm.at[pl.ds(off, K)], wc)
            pltpu.async_copy(lin_hbm.at[srcc], rows_v, sem).wait()

            def group_body(g, _):
                gb = g * 16
                d16 = dstc[pl.ds(gb, 16)]
                dl = d16 - lo
                inr = (dl >= 0) & (dl < RT)
                w16 = jnp.where(inr, wc[pl.ds(gb, 16)], 0.0)
                for e in range(16):
                    ge = gb + e
                    row0 = dstc[pl.ds(ge, 16)][0] - lo
                    own = (row0 >= 0) & (row0 < RT)
                    row = jnp.where(own, row0, 0)
                    wb = jnp.take(w16, jnp.full((16,), e, jnp.int32))
                    for ci in range(D // 16):
                        sl = pl.ds(ci * 16, 16)
                        acc[row, sl] = acc[row, sl] + rows_v[ge, sl] * wb
                return 0

            lax.fori_loop(0, K // 16, group_body, 0)
            return 0

        lax.fori_loop(0, nch, chunk_body, 0)
        pltpu.sync_copy(acc, out_hbm.at[pl.ds(lo, RT)])

    return k(lin, srcs, dsts, ws, bounds)


def _mm_first(x, wt, b, t0):
    def body(x_ref, wt_ref, b_ref, t_ref, lin_ref, hid_ref):
        xb = x_ref[...]
        lin_ref[...] = (
            jnp.dot(xb, wt_ref[...], preferred_element_type=jnp.float32)
            + b_ref[...])
        hid_ref[...] = xb * t_ref[0, 0]

    return pl.pallas_call(
        body,
        grid=(NP // BM,),
        in_specs=[
            pl.BlockSpec((BM, D), lambda i: (i, 0)),
            pl.BlockSpec((D, D), lambda i: (0, 0)),
            pl.BlockSpec((1, D), lambda i: (0, 0)),
            pl.BlockSpec((1, 1), lambda i: (0, 0)),
        ],
        out_specs=[
            pl.BlockSpec((BM, D), lambda i: (i, 0)),
            pl.BlockSpec((BM, D), lambda i: (i, 0)),
        ],
        out_shape=[
            jax.ShapeDtypeStruct((NP, D), jnp.float32),
            jax.ShapeDtypeStruct((NP, D), jnp.float32),
        ],
    )(x, wt, b, t0)


def _mm_mid(agg, hidden, wt, b, tcur):
    def body(a_ref, h_ref, wt_ref, b_ref, t_ref, lin_ref, hid_ref):
        h = jnp.maximum(a_ref[...], 0.0)
        hid_ref[...] = h_ref[...] + h * t_ref[0, 0]
        lin_ref[...] = (
            jnp.dot(h, wt_ref[...], preferred_element_type=jnp.float32)
            + b_ref[...])

    return pl.pallas_call(
        body,
        grid=(NP // BM,),
        in_specs=[
            pl.BlockSpec((BM, D), lambda i: (i, 0)),
            pl.BlockSpec((BM, D), lambda i: (i, 0)),
            pl.BlockSpec((D, D), lambda i: (0, 0)),
            pl.BlockSpec((1, D), lambda i: (0, 0)),
            pl.BlockSpec((1, 1), lambda i: (0, 0)),
        ],
        out_specs=[
            pl.BlockSpec((BM, D), lambda i: (i, 0)),
            pl.BlockSpec((BM, D), lambda i: (i, 0)),
        ],
        out_shape=[
            jax.ShapeDtypeStruct((NP, D), jnp.float32),
            jax.ShapeDtypeStruct((NP, D), jnp.float32),
        ],
    )(agg, hidden, wt, b, tcur)


def _mm_last(agg, hidden, tcur):
    def body(a_ref, h_ref, t_ref, out_ref):
        h = jnp.maximum(a_ref[...], 0.0)
        out_ref[...] = h_ref[...] + h * t_ref[0, 0]

    return pl.pallas_call(
        body,
        grid=(NP // BM,),
        in_specs=[
            pl.BlockSpec((BM, D), lambda i: (i, 0)),
            pl.BlockSpec((BM, D), lambda i: (i, 0)),
            pl.BlockSpec((1, 1), lambda i: (0, 0)),
        ],
        out_specs=pl.BlockSpec((BM, D), lambda i: (i, 0)),
        out_shape=jax.ShapeDtypeStruct((NP, D), jnp.float32),
    )(agg, hidden, tcur)


def kernel(x, edge_index, edge_weight, W0, b0, W1, b1, W2, b2, temp):
    src0 = edge_index[0].astype(jnp.int32)
    dst0 = edge_index[1].astype(jnp.int32)
    order = jnp.argsort(dst0)
    npad = ES - N_EDGES
    srcs = jnp.concatenate(
        [src0[order], jnp.zeros((npad,), jnp.int32)])
    dsts = jnp.concatenate(
        [dst0[order], jnp.full((npad,), NP - 1, jnp.int32)])
    ws = jnp.concatenate(
        [edge_weight[order], jnp.zeros((npad,), jnp.float32)])
    bnd = jnp.searchsorted(dsts, jnp.arange(0, NP + 1, RT, dtype=jnp.int32))
    bounds = jnp.concatenate(
        [bnd.astype(jnp.int32), jnp.zeros((48 - NW - 1,), jnp.int32)])
    xp = jnp.concatenate([x, jnp.zeros((NP - N_NODES, D), jnp.float32)])
    t = temp.reshape(4, 1)

    lin, hidden = _mm_first(xp, W0.T, b0.reshape(1, D), t[0:1])
    agg = _sc_agg(lin, srcs, dsts, ws, bounds)
    lin, hidden = _mm_mid(agg, hidden, W1.T, b1.reshape(1, D), t[1:2])
    agg = _sc_agg(lin, srcs, dsts, ws, bounds)
    lin, hidden = _mm_mid(agg, hidden, W2.T, b2.reshape(1, D), t[2:3])
    agg = _sc_agg(lin, srcs, dsts, ws, bounds)
    return _mm_last(agg, hidden, t[3:4])[:N_NODES]


# speedup vs baseline: 1.2994x; 1.2994x over previous
import functools

import jax
import jax.numpy as jnp
from jax import lax
from jax.experimental import pallas as pl
from jax.experimental.pallas import tpu as pltpu
from jax.experimental.pallas import tpu_sc as plsc

N_NODES = 10000
N_EDGES = 160000
D = 256

NP = 10240
NC = 2
NS = 16
NW = NC * NS
RT = NP // NW
K = 80
ES = N_EDGES + 2 * K
BM = 1024


def _sc_agg(lin, srcs, dsts, ws, bounds):
    mesh = plsc.VectorSubcoreMesh(core_axis_name="c", subcore_axis_name="s")

    @functools.partial(
        pl.kernel,
        out_type=jax.ShapeDtypeStruct((NP, D), jnp.float32),
        mesh=mesh,
        scratch_types=[
            pltpu.VMEM((48,), jnp.int32),
            pltpu.VMEM((2, K), jnp.int32),
            pltpu.VMEM((2, K), jnp.int32),
            pltpu.VMEM((2, K), jnp.float32),
            pltpu.VMEM((2, K, D), jnp.float32),
            pltpu.VMEM((RT, D), jnp.float32),
            pltpu.SemaphoreType.DMA,
            pltpu.SemaphoreType.DMA,
        ],
    )
    def k(lin_hbm, srcs_hbm, dsts_hbm, ws_hbm, bounds_hbm, out_hbm,
          bounds_v, srcc, dstc, wc, rbuf, acc, sem_e, sem_g):
        c = lax.axis_index("c")
        s = lax.axis_index("s")
        t = c * NS + s
        lo = t * RT
        zv = jnp.zeros((16,), jnp.float32)

        def zero_row(r, _):
            for ci in range(D // 16):
                acc[r, pl.ds(ci * 16, 16)] = zv
            return 0

        lax.fori_loop(0, RT, zero_row, 0)

        pltpu.sync_copy(bounds_hbm, bounds_v)
        b0 = bounds_v[pl.ds(t, 16)][0]
        b1 = bounds_v[pl.ds(t + 1, 16)][0]
        cb0 = b0 // K
        nch = (b1 + K - 1) // K - cb0

        def stage(jc, bb):
            off = (cb0 + jc) * K
            d1 = pltpu.async_copy(srcs_hbm.at[pl.ds(off, K)],
                                  srcc.at[bb], sem_e)
            d2 = pltpu.async_copy(dsts_hbm.at[pl.ds(off, K)],
                                  dstc.at[bb], sem_e)
            d3 = pltpu.async_copy(ws_hbm.at[pl.ds(off, K)],
                                  wc.at[bb], sem_e)
            d1.wait()
            d2.wait()
            d3.wait()

        def gather(bb):
            return pltpu.async_copy(
                lin_hbm.at[srcc.at[bb]], rbuf.at[bb], sem_g)

        @pl.when(nch > 0)
        def _():
            stage(0, 0)
            gather(0)

        def chunk_body(jc, _):
            bb = lax.rem(jc, 2)
            nb = 1 - bb
            stage(jc + 1, nb)
            pltpu.make_async_copy(
                lin_hbm.at[pl.ds(0, K)], rbuf.at[bb], sem_g).wait()
            gather(nb)

            def group_body(g, _):
                gb = g * 16
                d16 = dstc[bb, pl.ds(gb, 16)]
                dl = d16 - lo
                inr = (dl >= 0) & (dl < RT)
                w16 = jnp.where(inr, wc[bb, pl.ds(gb, 16)], 0.0)
                for e in range(16):
                    ge = gb + e
                    row0 = d16[e] - lo
                    own = (row0 >= 0) & (row0 < RT)
                    row = jnp.where(own, row0, 0)
                    wb = jnp.take(w16, jnp.full((16,), e, jnp.int32))
                    for ci in range(D // 16):
                        sl = pl.ds(ci * 16, 16)
                        acc[row, sl] = acc[row, sl] + rbuf[bb, ge, sl] * wb
                return 0

            lax.fori_loop(0, K // 16, group_body, 0)
            return 0

        lax.fori_loop(0, nch, chunk_body, 0)

        @pl.when(nch > 0)
        def _():
            pltpu.make_async_copy(
                lin_hbm.at[pl.ds(0, K)],
                rbuf.at[lax.rem(nch, 2)], sem_g).wait()

        pltpu.sync_copy(acc, out_hbm.at[pl.ds(lo, RT)])

    return k(lin, srcs, dsts, ws, bounds)


def _mm_first(x, wt, b, t0):
    def body(x_ref, wt_ref, b_ref, t_ref, lin_ref, hid_ref):
        xb = x_ref[...]
        lin_ref[...] = (
            jnp.dot(xb, wt_ref[...], preferred_element_type=jnp.float32)
            + b_ref[...])
        hid_ref[...] = xb * t_ref[0, 0]

    return pl.pallas_call(
        body,
        grid=(NP // BM,),
        in_specs=[
            pl.BlockSpec((BM, D), lambda i: (i, 0)),
            pl.BlockSpec((D, D), lambda i: (0, 0)),
            pl.BlockSpec((1, D), lambda i: (0, 0)),
            pl.BlockSpec((1, 1), lambda i: (0, 0)),
        ],
        out_specs=[
            pl.BlockSpec((BM, D), lambda i: (i, 0)),
            pl.BlockSpec((BM, D), lambda i: (i, 0)),
        ],
        out_shape=[
            jax.ShapeDtypeStruct((NP, D), jnp.float32),
            jax.ShapeDtypeStruct((NP, D), jnp.float32),
        ],
    )(x, wt, b, t0)


def _mm_mid(agg, hidden, wt, b, tcur):
    def body(a_ref, h_ref, wt_ref, b_ref, t_ref, lin_ref, hid_ref):
        h = jnp.maximum(a_ref[...], 0.0)
        hid_ref[...] = h_ref[...] + h * t_ref[0, 0]
        lin_ref[...] = (
            jnp.dot(h, wt_ref[...], preferred_element_type=jnp.float32)
            + b_ref[...])

    return pl.pallas_call(
        body,
        grid=(NP // BM,),
        in_specs=[
            pl.BlockSpec((BM, D), lambda i: (i, 0)),
            pl.BlockSpec((BM, D), lambda i: (i, 0)),
            pl.BlockSpec((D, D), lambda i: (0, 0)),
            pl.BlockSpec((1, D), lambda i: (0, 0)),
            pl.BlockSpec((1, 1), lambda i: (0, 0)),
        ],
        out_specs=[
            pl.BlockSpec((BM, D), lambda i: (i, 0)),
            pl.BlockSpec((BM, D), lambda i: (i, 0)),
        ],
        out_shape=[
            jax.ShapeDtypeStruct((NP, D), jnp.float32),
            jax.ShapeDtypeStruct((NP, D), jnp.float32),
        ],
    )(agg, hidden, wt, b, tcur)


def _mm_last(agg, hidden, tcur):
    def body(a_ref, h_ref, t_ref, out_ref):
        h = jnp.maximum(a_ref[...], 0.0)
        out_ref[...] = h_ref[...] + h * t_ref[0, 0]

    return pl.pallas_call(
        body,
        grid=(NP // BM,),
        in_specs=[
            pl.BlockSpec((BM, D), lambda i: (i, 0)),
            pl.BlockSpec((BM, D), lambda i: (i, 0)),
            pl.BlockSpec((1, 1), lambda i: (0, 0)),
        ],
        out_specs=pl.BlockSpec((BM, D), lambda i: (i, 0)),
        out_shape=jax.ShapeDtypeStruct((NP, D), jnp.float32),
    )(agg, hidden, tcur)


def kernel(x, edge_index, edge_weight, W0, b0, W1, b1, W2, b2, temp):
    src0 = edge_index[0].astype(jnp.int32)
    dst0 = edge_index[1].astype(jnp.int32)
    order = jnp.argsort(dst0)
    npad = ES - N_EDGES
    dsts_real = dst0[order]
    srcs = jnp.concatenate(
        [src0[order], jnp.zeros((npad,), jnp.int32)])
    dsts = jnp.concatenate(
        [dsts_real, jnp.full((npad,), NP - 1, jnp.int32)])
    ws = jnp.concatenate(
        [edge_weight[order], jnp.zeros((npad,), jnp.float32)])
    bnd = jnp.searchsorted(dsts_real,
                           jnp.arange(0, NP + 1, RT, dtype=jnp.int32))
    bounds = jnp.concatenate(
        [bnd.astype(jnp.int32), jnp.zeros((48 - NW - 1,), jnp.int32)])
    xp = jnp.concatenate([x, jnp.zeros((NP - N_NODES, D), jnp.float32)])
    t = temp.reshape(4, 1)

    lin, hidden = _mm_first(xp, W0.T, b0.reshape(1, D), t[0:1])
    agg = _sc_agg(lin, srcs, dsts, ws, bounds)
    lin, hidden = _mm_mid(agg, hidden, W1.T, b1.reshape(1, D), t[1:2])
    agg = _sc_agg(lin, srcs, dsts, ws, bounds)
    lin, hidden = _mm_mid(agg, hidden, W2.T, b2.reshape(1, D), t[2:3])
    agg = _sc_agg(lin, srcs, dsts, ws, bounds)
    return _mm_last(agg, hidden, t[3:4])[:N_NODES]


# speedup vs baseline: 2.8337x; 2.1807x over previous
import functools

import jax
import jax.numpy as jnp
from jax import lax
from jax.experimental import pallas as pl
from jax.experimental.pallas import tpu as pltpu
from jax.experimental.pallas import tpu_sc as plsc

N_NODES = 10000
N_EDGES = 160000
D = 256

NP = 10240
NC = 2
NS = 16
NW = NC * NS
RT = NP // NW
K = 80
ES = N_EDGES + 2 * K
BM = 1024


def _sc_agg(lin, srcs, dsts, ws, bounds):
    mesh = plsc.VectorSubcoreMesh(core_axis_name="c", subcore_axis_name="s")

    @functools.partial(
        pl.kernel,
        out_type=jax.ShapeDtypeStruct((NP, D), jnp.float32),
        mesh=mesh,
        scratch_types=[
            pltpu.VMEM((48,), jnp.int32),
            pltpu.VMEM((2, K), jnp.int32),
            pltpu.VMEM((2, K), jnp.int32),
            pltpu.VMEM((2, K), jnp.float32),
            pltpu.VMEM((2, K, D), jnp.float32),
            pltpu.VMEM((RT, D), jnp.float32),
            pltpu.SemaphoreType.DMA,
            pltpu.SemaphoreType.DMA,
        ],
    )
    def k(lin_hbm, srcs_hbm, dsts_hbm, ws_hbm, bounds_hbm, out_hbm,
          bounds_v, srcc, dstc, wc, rbuf, acc, sem_e, sem_g):
        c = lax.axis_index("c")
        s = lax.axis_index("s")
        t = c * NS + s
        lo = t * RT
        zv = jnp.zeros((16,), jnp.float32)

        def zero_row(r, _):
            for ci in range(D // 16):
                acc[r, pl.ds(ci * 16, 16)] = zv
            return 0

        lax.fori_loop(0, RT, zero_row, 0)

        pltpu.sync_copy(bounds_hbm, bounds_v)
        b0 = bounds_v[pl.ds(t, 16)][0]
        b1 = bounds_v[pl.ds(t + 1, 16)][0]
        cb0 = b0 // K
        nch = (b1 + K - 1) // K - cb0

        def stage(jc, bb):
            off = (cb0 + jc) * K
            d1 = pltpu.async_copy(srcs_hbm.at[pl.ds(off, K)],
                                  srcc.at[bb], sem_e)
            d2 = pltpu.async_copy(dsts_hbm.at[pl.ds(off, K)],
                                  dstc.at[bb], sem_e)
            d3 = pltpu.async_copy(ws_hbm.at[pl.ds(off, K)],
                                  wc.at[bb], sem_e)
            d1.wait()
            d2.wait()
            d3.wait()

        def gather(bb):
            return pltpu.async_copy(
                lin_hbm.at[srcc.at[bb]], rbuf.at[bb], sem_g)

        @pl.when(nch > 0)
        def _():
            stage(0, 0)
            gather(0)

        nvr = D // 16
        init = (jnp.int32(-1),) + tuple(zv for _ in range(nvr))

        def chunk_body(jc, carry):
            bb = lax.rem(jc, 2)
            nb = 1 - bb
            stage(jc + 1, nb)
            pltpu.make_async_copy(
                lin_hbm.at[pl.ds(0, K)], rbuf.at[bb], sem_g).wait()
            gather(nb)

            def group_body(g, carry):
                gb = g * 16
                d16 = dstc[bb, pl.ds(gb, 16)]
                dl = d16 - lo
                inr = (dl >= 0) & (dl < RT)
                w16 = jnp.where(inr, wc[bb, pl.ds(gb, 16)], 0.0)
                prev = carry[0]
                regs = list(carry[1:])
                for e in range(16):
                    ge = gb + e
                    row0 = d16[e] - lo
                    own = (row0 >= 0) & (row0 < RT)
                    row = jnp.where(own, row0, 0)
                    wb = jnp.take(w16, jnp.full((16,), e, jnp.int32))
                    same = row == prev

                    @pl.when(jnp.logical_not(same) & (prev >= 0))
                    def _(regs=regs, prev=prev):
                        for ci in range(nvr):
                            sl = pl.ds(ci * 16, 16)
                            acc[prev, sl] = acc[prev, sl] + regs[ci]

                    for ci in range(nvr):
                        contrib = rbuf[bb, ge, pl.ds(ci * 16, 16)] * wb
                        regs[ci] = jnp.where(same, regs[ci] + contrib, contrib)
                    prev = row
                return (prev,) + tuple(regs)

            return lax.fori_loop(0, K // 16, group_body, carry)

        fin = lax.fori_loop(0, nch, chunk_body, init)

        @pl.when(fin[0] >= 0)
        def _():
            for ci in range(nvr):
                sl = pl.ds(ci * 16, 16)
                acc[fin[0], sl] = acc[fin[0], sl] + fin[1 + ci]

        @pl.when(nch > 0)
        def _():
            pltpu.make_async_copy(
                lin_hbm.at[pl.ds(0, K)],
                rbuf.at[lax.rem(nch, 2)], sem_g).wait()

        pltpu.sync_copy(acc, out_hbm.at[pl.ds(lo, RT)])

    return k(lin, srcs, dsts, ws, bounds)


def _mm_first(x, wt, b, t0):
    def body(x_ref, wt_ref, b_ref, t_ref, lin_ref, hid_ref):
        xb = x_ref[...]
        lin_ref[...] = (
            jnp.dot(xb, wt_ref[...], preferred_element_type=jnp.float32)
            + b_ref[...])
        hid_ref[...] = xb * t_ref[0, 0]

    return pl.pallas_call(
        body,
        grid=(NP // BM,),
        in_specs=[
            pl.BlockSpec((BM, D), lambda i: (i, 0)),
            pl.BlockSpec((D, D), lambda i: (0, 0)),
            pl.BlockSpec((1, D), lambda i: (0, 0)),
            pl.BlockSpec((1, 1), lambda i: (0, 0)),
        ],
        out_specs=[
            pl.BlockSpec((BM, D), lambda i: (i, 0)),
            pl.BlockSpec((BM, D), lambda i: (i, 0)),
        ],
        out_shape=[
            jax.ShapeDtypeStruct((NP, D), jnp.float32),
            jax.ShapeDtypeStruct((NP, D), jnp.float32),
        ],
    )(x, wt, b, t0)


def _mm_mid(agg, hidden, wt, b, tcur):
    def body(a_ref, h_ref, wt_ref, b_ref, t_ref, lin_ref, hid_ref):
        h = jnp.maximum(a_ref[...], 0.0)
        hid_ref[...] = h_ref[...] + h * t_ref[0, 0]
        lin_ref[...] = (
            jnp.dot(h, wt_ref[...], preferred_element_type=jnp.float32)
            + b_ref[...])

    return pl.pallas_call(
        body,
        grid=(NP // BM,),
        in_specs=[
            pl.BlockSpec((BM, D), lambda i: (i, 0)),
            pl.BlockSpec((BM, D), lambda i: (i, 0)),
            pl.BlockSpec((D, D), lambda i: (0, 0)),
            pl.BlockSpec((1, D), lambda i: (0, 0)),
            pl.BlockSpec((1, 1), lambda i: (0, 0)),
        ],
        out_specs=[
            pl.BlockSpec((BM, D), lambda i: (i, 0)),
            pl.BlockSpec((BM, D), lambda i: (i, 0)),
        ],
        out_shape=[
            jax.ShapeDtypeStruct((NP, D), jnp.float32),
            jax.ShapeDtypeStruct((NP, D), jnp.float32),
        ],
    )(agg, hidden, wt, b, tcur)


def _mm_last(agg, hidden, tcur):
    def body(a_ref, h_ref, t_ref, out_ref):
        h = jnp.maximum(a_ref[...], 0.0)
        out_ref[...] = h_ref[...] + h * t_ref[0, 0]

    return pl.pallas_call(
        body,
        grid=(NP // BM,),
        in_specs=[
            pl.BlockSpec((BM, D), lambda i: (i, 0)),
            pl.BlockSpec((BM, D), lambda i: (i, 0)),
            pl.BlockSpec((1, 1), lambda i: (0, 0)),
        ],
        out_specs=pl.BlockSpec((BM, D), lambda i: (i, 0)),
        out_shape=jax.ShapeDtypeStruct((NP, D), jnp.float32),
    )(agg, hidden, tcur)


def kernel(x, edge_index, edge_weight, W0, b0, W1, b1, W2, b2, temp):
    src0 = edge_index[0].astype(jnp.int32)
    dst0 = edge_index[1].astype(jnp.int32)
    order = jnp.argsort(dst0)
    npad = ES - N_EDGES
    dsts_real = dst0[order]
    srcs = jnp.concatenate(
        [src0[order], jnp.zeros((npad,), jnp.int32)])
    dsts = jnp.concatenate(
        [dsts_real, jnp.full((npad,), NP - 1, jnp.int32)])
    ws = jnp.concatenate(
        [edge_weight[order], jnp.zeros((npad,), jnp.float32)])
    bnd = jnp.searchsorted(dsts_real,
                           jnp.arange(0, NP + 1, RT, dtype=jnp.int32))
    bounds = jnp.concatenate(
        [bnd.astype(jnp.int32), jnp.zeros((48 - NW - 1,), jnp.int32)])
    xp = jnp.concatenate([x, jnp.zeros((NP - N_NODES, D), jnp.float32)])
    t = temp.reshape(4, 1)

    lin, hidden = _mm_first(xp, W0.T, b0.reshape(1, D), t[0:1])
    agg = _sc_agg(lin, srcs, dsts, ws, bounds)
    lin, hidden = _mm_mid(agg, hidden, W1.T, b1.reshape(1, D), t[1:2])
    agg = _sc_agg(lin, srcs, dsts, ws, bounds)
    lin, hidden = _mm_mid(agg, hidden, W2.T, b2.reshape(1, D), t[2:3])
    agg = _sc_agg(lin, srcs, dsts, ws, bounds)
    return _mm_last(agg, hidden, t[3:4])[:N_NODES]
